# baseline (speedup 1.0000x reference)
import jax
import jax.numpy as jnp
from jax import lax
from jax.experimental import pallas as pl
from jax.experimental.pallas import tpu as pltpu
from jax.experimental.pallas import tpu_sc as plsc

N = 10000
IN_F = 768
HID = 128

NC = 2
NS = 16
LANES = 16
CH = 128

N_T = 10240
RPT = N_T // NS
RB = 512
NBLK = N_T // RB


def _fill2d(ref, rows, cols, val):
    def body(i, _):
        r = i // (cols // LANES)
        j = i % (cols // LANES)
        ref[r, pl.ds(j * LANES, LANES)] = jnp.full((LANES,), val, jnp.float32)
        return 0
    lax.fori_loop(0, rows * (cols // LANES), body, 0, unroll=False)


def _fill_iota(ref, n, base):
    def body(i, _):
        ref[pl.ds(i * LANES, LANES)] = lax.iota(jnp.int32, LANES) + base + i * LANES
        return 0
    lax.fori_loop(0, n // LANES, body, 0, unroll=False)


def _zero_own_rows(acc, rv, zb_v, s):
    def zb(i, _):
        _fill_iota(rv, CH, s * RPT + i * CH)
        pltpu.sync_copy(zb_v, acc.at[rv])
        return 0
    lax.fori_loop(0, RPT // CH, zb, 0, unroll=False)


def _readout_own_rows(acc, rv, buf_v, o_h, c_sel, s, sem):
    def ob(i, _):
        r = s * RPT + i * CH
        _fill_iota(rv, CH, r)
        pltpu.async_copy(acc.at[rv], buf_v, sem).wait()
        pltpu.sync_copy(buf_v, o_h.at[pl.ds(r, CH)])
        return 0
    lax.fori_loop(0, RPT // CH, ob, 0, unroll=False)


NB = 2
NB_AGG = 1
BATCH = 32


def _make_deg(e_pad):
    chunks = e_pad // (NS * CH)
    mesh = plsc.VectorSubcoreMesh(core_axis_name="c", subcore_axis_name="s",
                                  num_cores=NC, num_subcores=NS)

    def body(src_h, dst_h, dsrc_h, ddst_h, rv, idx2d, ones_v, zb_v, acc,
             s0, s1):
        c = lax.axis_index("c")
        s = lax.axis_index("s")
        sems = [s0, s1]
        _fill2d(ones_v, CH, HID, 1.0)
        _fill2d(zb_v, CH, HID, 0.0)
        _zero_own_rows(acc, rv, zb_v, s)
        plsc.subcore_barrier()

        row0 = s * chunks

        def run(i_h, o_h):
            def batch(bi, _):
                pltpu.sync_copy(i_h.at[pl.ds(row0 + bi * BATCH, BATCH)], idx2d)

                def grp(g, _):
                    i0 = g * NB
                    descs = [
                        pltpu.async_copy(ones_v, acc.at[idx2d.at[i0 + b]],
                                         sems[b], add=True)
                        for b in range(NB)
                    ]
                    for d in descs:
                        d.wait()
                    return 0
                lax.fori_loop(0, BATCH // NB, grp, 0, unroll=False)
                return 0
            lax.fori_loop(0, chunks // BATCH, batch, 0, unroll=False)
            plsc.subcore_barrier()
            _readout_own_rows(acc, rv, zb_v, o_h, c, s, sems[0])

        @pl.when(c == 0)
        def _():
            run(src_h, dsrc_h)

        @pl.when(c == 1)
        def _():
            run(dst_h, ddst_h)

    return pl.kernel(
        body,
        out_type=(
            jax.ShapeDtypeStruct((N_T, HID), jnp.float32),
            jax.ShapeDtypeStruct((N_T, HID), jnp.float32),
        ),
        mesh=mesh,
        scratch_types=(
            pltpu.VMEM((CH,), jnp.int32),
            pltpu.VMEM((BATCH, CH), jnp.int32),
            pltpu.VMEM((CH, HID), jnp.float32),
            pltpu.VMEM((CH, HID), jnp.float32),
            pltpu.VMEM_SHARED((N_T, HID), jnp.float32),
            pltpu.SemaphoreType.DMA,
            pltpu.SemaphoreType.DMA,
        ),
    )


def _make_agg(e_pad):
    chunks = e_pad // (NS * CH)
    mesh = plsc.VectorSubcoreMesh(core_axis_name="c", subcore_axis_name="s",
                                  num_cores=NC, num_subcores=NS)

    def body(t0_h, t1_h, i0_h, i1_h, dst_h, out0_h, out1_h,
             rv, iv0, iv1, ov0, ov1, r0, r1, acc, s0, s1):
        c = lax.axis_index("c")
        s = lax.axis_index("s")
        rows = [r0, r1][:NB_AGG]
        ivs = [iv0, iv1][:NB_AGG]
        ovs = [ov0, ov1][:NB_AGG]
        sems = [s0, s1][:NB_AGG]
        _fill2d(r0, CH, HID, 0.0)
        _zero_own_rows(acc, rv, r0, s)
        plsc.subcore_barrier()

        base0 = s * chunks * CH

        def run(t_h, i_h, o_h):
            def grp(g, _):
                i0 = g * NB_AGG
                descs = []
                for b in range(NB_AGG):
                    base = base0 + (i0 + b) * CH
                    pltpu.sync_copy(i_h.at[pl.ds(base, CH)], ivs[b])
                    pltpu.sync_copy(dst_h.at[pl.ds(base, CH)], ovs[b])
                    descs.append(
                        pltpu.async_copy(t_h.at[ivs[b]], rows[b], sems[b]))
                for b in range(NB_AGG):
                    descs[b].wait()
                    pltpu.sync_copy(rows[b], acc.at[ovs[b]], add=True)
                return 0
            lax.fori_loop(0, chunks // NB_AGG, grp, 0, unroll=False)
            plsc.subcore_barrier()
            _readout_own_rows(acc, rv, r0, o_h, c, s, sems[0])

        @pl.when(c == 0)
        def _():
            run(t0_h, i0_h, out0_h)

        @pl.when(c == 1)
        def _():
            run(t1_h, i1_h, out1_h)

    return pl.kernel(
        body,
        out_type=(
            jax.ShapeDtypeStruct((N_T, HID), jnp.float32),
            jax.ShapeDtypeStruct((N_T, HID), jnp.float32),
        ),
        mesh=mesh,
        scratch_types=(
            pltpu.VMEM((CH,), jnp.int32),
            pltpu.VMEM((CH,), jnp.int32),
            pltpu.VMEM((CH,), jnp.int32),
            pltpu.VMEM((CH,), jnp.int32),
            pltpu.VMEM((CH,), jnp.int32),
            pltpu.VMEM((CH, HID), jnp.float32),
            pltpu.VMEM((CH, HID), jnp.float32),
            pltpu.VMEM_SHARED((N_T, HID), jnp.float32),
            pltpu.SemaphoreType.DMA,
            pltpu.SemaphoreType.DMA,
        ),
    )


def _make_perm():
    nch = N_T // CH
    nw = NC * NS
    mesh = plsc.VectorSubcoreMesh(core_axis_name="c", subcore_axis_name="s",
                                  num_cores=NC, num_subcores=NS)

    def body(t_h, pidx_h, out_h, iv, rows_v, sem):
        c = lax.axis_index("c")
        s = lax.axis_index("s")
        w = c * NS + s

        def cb(k, _):
            ch = w + nw * k

            @pl.when(ch < nch)
            def _():
                base = ch * CH
                pltpu.sync_copy(pidx_h.at[pl.ds(base, CH)], iv)
                pltpu.async_copy(t_h.at[iv], rows_v, sem).wait()
                pltpu.sync_copy(rows_v, out_h.at[pl.ds(base, CH)])
            return 0
        lax.fori_loop(0, (nch + nw - 1) // nw, cb, 0, unroll=False)

    return pl.kernel(
        body,
        out_type=(jax.ShapeDtypeStruct((N_T, HID), jnp.float32),),
        mesh=mesh,
        scratch_types=(
            pltpu.VMEM((CH,), jnp.int32),
            pltpu.VMEM((CH, HID), jnp.float32),
            pltpu.SemaphoreType.DMA,
        ),
    )


def _enc0_body(x_ref, w_ref, h_ref):
    h_ref[...] = jnp.dot(x_ref[...], w_ref[...],
                         preferred_element_type=jnp.float32)


def _enc0(x_p, w0):
    return pl.pallas_call(
        _enc0_body,
        grid=(NBLK,),
        in_specs=[
            pl.BlockSpec((RB, IN_F), lambda i: (i, 0)),
            pl.BlockSpec((IN_F, HID), lambda i: (0, 0)),
        ],
        out_specs=pl.BlockSpec((RB, HID), lambda i: (i, 0)),
        out_shape=jax.ShapeDtypeStruct((N_T, HID), jnp.float32),
    )(x_p, w0)


def _scale_body(h_ref, hp_ref, dsrc_ref, tpos_ref, tneg_ref):
    r_out = lax.rsqrt(jnp.maximum(dsrc_ref[...][:, :1], 1.0))
    tpos_ref[...] = h_ref[...] * r_out
    tneg_ref[...] = hp_ref[...] * r_out


def _scale(h0, h0p, dsrc2d):
    return pl.pallas_call(
        _scale_body,
        grid=(NBLK,),
        in_specs=[
            pl.BlockSpec((RB, HID), lambda i: (i, 0)),
            pl.BlockSpec((RB, HID), lambda i: (i, 0)),
            pl.BlockSpec((RB, HID), lambda i: (i, 0)),
        ],
        out_specs=[
            pl.BlockSpec((RB, HID), lambda i: (i, 0)),
            pl.BlockSpec((RB, HID), lambda i: (i, 0)),
        ],
        out_shape=[
            jax.ShapeDtypeStruct((N_T, HID), jnp.float32),
            jax.ShapeDtypeStruct((N_T, HID), jnp.float32),
        ],
    )(h0, h0p, dsrc2d)


def _mid_body(ap_ref, an_ref, ddst_ref, dsrc_ref, b0_ref, w1_ref,
              up_ref, un_ref):
    rin = lax.rsqrt(jnp.maximum(ddst_ref[...][:, :1], 1.0))
    rout = lax.rsqrt(jnp.maximum(dsrc_ref[...][:, :1], 1.0))
    b0 = b0_ref[...]
    w1 = w1_ref[...]
    zp = jnp.maximum(ap_ref[...] * rin + b0, 0.0)
    zn = jnp.maximum(an_ref[...] * rin + b0, 0.0)
    up_ref[...] = jnp.dot(zp, w1, preferred_element_type=jnp.float32) * rout
    un_ref[...] = jnp.dot(zn, w1, preferred_element_type=jnp.float32) * rout


def _mid(ap, an, ddst2d, dsrc2d, b0, w1):
    return pl.pallas_call(
        _mid_body,
        grid=(NBLK,),
        in_specs=[
            pl.BlockSpec((RB, HID), lambda i: (i, 0)),
            pl.BlockSpec((RB, HID), lambda i: (i, 0)),
            pl.BlockSpec((RB, HID), lambda i: (i, 0)),
            pl.BlockSpec((RB, HID), lambda i: (i, 0)),
            pl.BlockSpec((1, HID), lambda i: (0, 0)),
            pl.BlockSpec((HID, HID), lambda i: (0, 0)),
        ],
        out_specs=[
            pl.BlockSpec((RB, HID), lambda i: (i, 0)),
            pl.BlockSpec((RB, HID), lambda i: (i, 0)),
        ],
        out_shape=[
            jax.ShapeDtypeStruct((N_T, HID), jnp.float32),
            jax.ShapeDtypeStruct((N_T, HID), jnp.float32),
        ],
    )(ap, an, ddst2d, dsrc2d, b0, w1)


def _softplus(v):
    return jnp.maximum(v, 0.0) + jnp.log(1.0 + jnp.exp(-jnp.abs(v)))


def _head_body(ap_ref, an_ref, ddst_ref, b1_ref, wd_ref, out_ref):
    b1 = b1_ref[...]

    def chunk(i):
        rin = lax.rsqrt(
            jnp.maximum(ddst_ref[pl.ds(i * RB, RB), :][:, :1], 1.0))
        pos = ap_ref[pl.ds(i * RB, RB), :] * rin + b1
        neg = an_ref[pl.ds(i * RB, RB), :] * rin + b1
        rowid = lax.broadcasted_iota(jnp.int32, (RB, 1), 0) + i * RB
        m = (rowid < N).astype(jnp.float32)
        return pos, neg, m

    def body1(i, colsum):
        pos, _, m = chunk(i)
        return colsum + jnp.sum(pos * m, axis=0, keepdims=True)

    colsum = lax.fori_loop(0, NBLK, body1, jnp.zeros((1, HID), jnp.float32))
    summary = 1.0 / (1.0 + jnp.exp(-colsum / N))
    wd = wd_ref[...]
    ws = lax.dot_general(summary, wd, (((1,), (1,)), ((), ())),
                         preferred_element_type=jnp.float32)

    def body2(i, carry):
        l1s, l2s = carry
        pos, neg, m = chunk(i)
        psc = lax.dot_general(pos, ws, (((1,), (1,)), ((), ())),
                              preferred_element_type=jnp.float32)
        nsc = lax.dot_general(neg, ws, (((1,), (1,)), ((), ())),
                              preferred_element_type=jnp.float32)
        l1s = l1s + jnp.sum(_softplus(-psc) * m)
        l2s = l2s + jnp.sum(_softplus(nsc) * m)
        return l1s, l2s

    l1s, l2s = lax.fori_loop(
        0, NBLK, body2, (jnp.float32(0.0), jnp.float32(0.0)))
    out_ref[...] = jnp.reshape((l1s + l2s) / jnp.float32(N), (1, 1))


def _head(ap, an, ddst2d, b1, wd):
    return pl.pallas_call(
        _head_body,
        out_shape=jax.ShapeDtypeStruct((1, 1), jnp.float32),
    )(ap, an, ddst2d, b1, wd)


@jax.jit
def kernel(x, edge_index, W0, b0, W1, b1, Wd, Wc, bc):
    del Wc, bc
    e = edge_index.shape[1]
    unit = NS * CH * BATCH
    e_pad = ((e + unit - 1) // unit) * unit

    src = edge_index[0].astype(jnp.int32)
    dst = edge_index[1].astype(jnp.int32)
    pad = jnp.full((e_pad - e,), N, jnp.int32)
    src_p = jnp.concatenate([src, pad])
    dst_p = jnp.concatenate([dst, pad])
    src2d = jnp.reshape(src_p, (e_pad // CH, CH))
    dst2d = jnp.reshape(dst_p, (e_pad // CH, CH))
    perm = jax.random.permutation(jax.random.key(42), N).astype(jnp.int32)
    perm_p = jnp.concatenate([perm, jnp.arange(N, N_T, dtype=jnp.int32)])
    x_p = jnp.pad(x, ((0, N_T - N), (0, 0)))

    dsrc2d, ddst2d = _make_deg(e_pad)(src2d, dst2d)
    h0 = _enc0(x_p, W0)
    (h0p,) = _make_perm()(h0, perm_p)
    tpos, tneg = _scale(h0, h0p, dsrc2d)
    agg = _make_agg(e_pad)
    acc1p, acc1n = agg(tpos, tneg, src_p, src_p, dst_p)
    up, un = _mid(acc1p, acc1n, ddst2d, dsrc2d, jnp.reshape(b0, (1, HID)), W1)
    acc2p, acc2n = agg(up, un, src_p, src_p, dst_p)
    res = _head(acc2p, acc2n, ddst2d, jnp.reshape(b1, (1, HID)), Wd)
    return res[0, 0]


# speedup vs baseline: 1.0013x; 1.0013x over previous
import jax
import jax.numpy as jnp
from jax import lax
from jax.experimental import pallas as pl
from jax.experimental.pallas import tpu as pltpu
from jax.experimental.pallas import tpu_sc as plsc

N = 10000
IN_F = 768
HID = 128

NC = 2
NS = 16
LANES = 16
CH = 128

N_T = 10240
RPT = N_T // NS
RB = 512
NBLK = N_T // RB


def _fill2d(ref, rows, cols, val):
    def body(i, _):
        r = i // (cols // LANES)
        j = i % (cols // LANES)
        ref[r, pl.ds(j * LANES, LANES)] = jnp.full((LANES,), val, jnp.float32)
        return 0
    lax.fori_loop(0, rows * (cols // LANES), body, 0, unroll=False)


def _fill_iota(ref, n, base):
    def body(i, _):
        ref[pl.ds(i * LANES, LANES)] = lax.iota(jnp.int32, LANES) + base + i * LANES
        return 0
    lax.fori_loop(0, n // LANES, body, 0, unroll=False)


def _zero_own_rows(acc, rv, zb_v, s):
    def zb(i, _):
        _fill_iota(rv, CH, s * RPT + i * CH)
        pltpu.sync_copy(zb_v, acc.at[rv])
        return 0
    lax.fori_loop(0, RPT // CH, zb, 0, unroll=False)


def _readout_own_rows(acc, rv, buf_v, o_h, c_sel, s, sem):
    def ob(i, _):
        r = s * RPT + i * CH
        _fill_iota(rv, CH, r)
        pltpu.async_copy(acc.at[rv], buf_v, sem).wait()
        pltpu.sync_copy(buf_v, o_h.at[pl.ds(r, CH)])
        return 0
    lax.fori_loop(0, RPT // CH, ob, 0, unroll=False)


NB = 2
NB_AGG = 1
BATCH = 32


def _make_deg(e_pad):
    chunks = e_pad // (NS * CH)
    mesh = plsc.VectorSubcoreMesh(core_axis_name="c", subcore_axis_name="s",
                                  num_cores=NC, num_subcores=NS)

    def body(src_h, dst_h, dsrc_h, ddst_h, rv, idx2d, ones_v, zb_v, acc,
             s0, s1):
        c = lax.axis_index("c")
        s = lax.axis_index("s")
        sems = [s0, s1]
        _fill2d(ones_v, CH, HID, 1.0)
        _fill2d(zb_v, CH, HID, 0.0)
        _zero_own_rows(acc, rv, zb_v, s)
        plsc.subcore_barrier()

        row0 = s * chunks

        def run(i_h, o_h):
            def batch(bi, _):
                pltpu.sync_copy(i_h.at[pl.ds(row0 + bi * BATCH, BATCH)], idx2d)

                def grp(g, _):
                    i0 = g * NB
                    descs = [
                        pltpu.async_copy(ones_v, acc.at[idx2d.at[i0 + b]],
                                         sems[b], add=True)
                        for b in range(NB)
                    ]
                    for d in descs:
                        d.wait()
                    return 0
                lax.fori_loop(0, BATCH // NB, grp, 0, unroll=False)
                return 0
            lax.fori_loop(0, chunks // BATCH, batch, 0, unroll=False)
            plsc.subcore_barrier()
            _readout_own_rows(acc, rv, zb_v, o_h, c, s, sems[0])

        @pl.when(c == 0)
        def _():
            run(src_h, dsrc_h)

        @pl.when(c == 1)
        def _():
            run(dst_h, ddst_h)

    return pl.kernel(
        body,
        out_type=(
            jax.ShapeDtypeStruct((N_T, HID), jnp.float32),
            jax.ShapeDtypeStruct((N_T, HID), jnp.float32),
        ),
        mesh=mesh,
        scratch_types=(
            pltpu.VMEM((CH,), jnp.int32),
            pltpu.VMEM((BATCH, CH), jnp.int32),
            pltpu.VMEM((CH, HID), jnp.float32),
            pltpu.VMEM((CH, HID), jnp.float32),
            pltpu.VMEM_SHARED((N_T, HID), jnp.float32),
            pltpu.SemaphoreType.DMA,
            pltpu.SemaphoreType.DMA,
        ),
    )


def _make_agg(e_pad):
    chunks = e_pad // (NS * CH)
    mesh = plsc.VectorSubcoreMesh(core_axis_name="c", subcore_axis_name="s",
                                  num_cores=NC, num_subcores=NS)

    def body(t0_h, t1_h, i0_h, i1_h, dst_h, out0_h, out1_h,
             iv, ov, rv, rows_v, zb_v, acc, sem):
        c = lax.axis_index("c")
        s = lax.axis_index("s")
        _fill2d(zb_v, CH, HID, 0.0)
        _zero_own_rows(acc, rv, zb_v, s)
        plsc.subcore_barrier()

        base0 = s * chunks * CH

        def run(t_h, i_h, o_h):
            def cb(i, _):
                base = base0 + i * CH
                pltpu.sync_copy(i_h.at[pl.ds(base, CH)], iv)
                pltpu.sync_copy(dst_h.at[pl.ds(base, CH)], ov)
                pltpu.async_copy(t_h.at[iv], rows_v, sem).wait()
                pltpu.sync_copy(rows_v, acc.at[ov], add=True)
                return 0
            lax.fori_loop(0, chunks, cb, 0, unroll=False)
            plsc.subcore_barrier()
            _readout_own_rows(acc, rv, zb_v, o_h, c, s, sem)

        @pl.when(c == 0)
        def _():
            run(t0_h, i0_h, out0_h)

        @pl.when(c == 1)
        def _():
            run(t1_h, i1_h, out1_h)

    return pl.kernel(
        body,
        out_type=(
            jax.ShapeDtypeStruct((N_T, HID), jnp.float32),
            jax.ShapeDtypeStruct((N_T, HID), jnp.float32),
        ),
        mesh=mesh,
        scratch_types=(
            pltpu.VMEM((CH,), jnp.int32),
            pltpu.VMEM((CH,), jnp.int32),
            pltpu.VMEM((CH,), jnp.int32),
            pltpu.VMEM((CH, HID), jnp.float32),
            pltpu.VMEM((CH, HID), jnp.float32),
            pltpu.VMEM_SHARED((N_T, HID), jnp.float32),
            pltpu.SemaphoreType.DMA,
        ),
    )


def _make_perm():
    nch = N_T // CH
    nw = NC * NS
    mesh = plsc.VectorSubcoreMesh(core_axis_name="c", subcore_axis_name="s",
                                  num_cores=NC, num_subcores=NS)

    def body(t_h, pidx_h, out_h, iv, rows_v, sem):
        c = lax.axis_index("c")
        s = lax.axis_index("s")
        w = c * NS + s

        def cb(k, _):
            ch = w + nw * k

            @pl.when(ch < nch)
            def _():
                base = ch * CH
                pltpu.sync_copy(pidx_h.at[pl.ds(base, CH)], iv)
                pltpu.async_copy(t_h.at[iv], rows_v, sem).wait()
                pltpu.sync_copy(rows_v, out_h.at[pl.ds(base, CH)])
            return 0
        lax.fori_loop(0, (nch + nw - 1) // nw, cb, 0, unroll=False)

    return pl.kernel(
        body,
        out_type=(jax.ShapeDtypeStruct((N_T, HID), jnp.float32),),
        mesh=mesh,
        scratch_types=(
            pltpu.VMEM((CH,), jnp.int32),
            pltpu.VMEM((CH, HID), jnp.float32),
            pltpu.SemaphoreType.DMA,
        ),
    )


def _enc0_body(x_ref, w_ref, h_ref):
    h_ref[...] = jnp.dot(x_ref[...], w_ref[...],
                         preferred_element_type=jnp.float32)


def _enc0(x_p, w0):
    return pl.pallas_call(
        _enc0_body,
        grid=(NBLK,),
        in_specs=[
            pl.BlockSpec((RB, IN_F), lambda i: (i, 0)),
            pl.BlockSpec((IN_F, HID), lambda i: (0, 0)),
        ],
        out_specs=pl.BlockSpec((RB, HID), lambda i: (i, 0)),
        out_shape=jax.ShapeDtypeStruct((N_T, HID), jnp.float32),
    )(x_p, w0)


def _scale_body(h_ref, hp_ref, dsrc_ref, tpos_ref, tneg_ref):
    r_out = lax.rsqrt(jnp.maximum(dsrc_ref[...][:, :1], 1.0))
    tpos_ref[...] = h_ref[...] * r_out
    tneg_ref[...] = hp_ref[...] * r_out


def _scale(h0, h0p, dsrc2d):
    return pl.pallas_call(
        _scale_body,
        grid=(NBLK,),
        in_specs=[
            pl.BlockSpec((RB, HID), lambda i: (i, 0)),
            pl.BlockSpec((RB, HID), lambda i: (i, 0)),
            pl.BlockSpec((RB, HID), lambda i: (i, 0)),
        ],
        out_specs=[
            pl.BlockSpec((RB, HID), lambda i: (i, 0)),
            pl.BlockSpec((RB, HID), lambda i: (i, 0)),
        ],
        out_shape=[
            jax.ShapeDtypeStruct((N_T, HID), jnp.float32),
            jax.ShapeDtypeStruct((N_T, HID), jnp.float32),
        ],
    )(h0, h0p, dsrc2d)


def _mid_body(ap_ref, an_ref, ddst_ref, dsrc_ref, b0_ref, w1_ref,
              up_ref, un_ref):
    rin = lax.rsqrt(jnp.maximum(ddst_ref[...][:, :1], 1.0))
    rout = lax.rsqrt(jnp.maximum(dsrc_ref[...][:, :1], 1.0))
    b0 = b0_ref[...]
    w1 = w1_ref[...]
    zp = jnp.maximum(ap_ref[...] * rin + b0, 0.0)
    zn = jnp.maximum(an_ref[...] * rin + b0, 0.0)
    up_ref[...] = jnp.dot(zp, w1, preferred_element_type=jnp.float32) * rout
    un_ref[...] = jnp.dot(zn, w1, preferred_element_type=jnp.float32) * rout


def _mid(ap, an, ddst2d, dsrc2d, b0, w1):
    return pl.pallas_call(
        _mid_body,
        grid=(NBLK,),
        in_specs=[
            pl.BlockSpec((RB, HID), lambda i: (i, 0)),
            pl.BlockSpec((RB, HID), lambda i: (i, 0)),
            pl.BlockSpec((RB, HID), lambda i: (i, 0)),
            pl.BlockSpec((RB, HID), lambda i: (i, 0)),
            pl.BlockSpec((1, HID), lambda i: (0, 0)),
            pl.BlockSpec((HID, HID), lambda i: (0, 0)),
        ],
        out_specs=[
            pl.BlockSpec((RB, HID), lambda i: (i, 0)),
            pl.BlockSpec((RB, HID), lambda i: (i, 0)),
        ],
        out_shape=[
            jax.ShapeDtypeStruct((N_T, HID), jnp.float32),
            jax.ShapeDtypeStruct((N_T, HID), jnp.float32),
        ],
    )(ap, an, ddst2d, dsrc2d, b0, w1)


def _softplus(v):
    return jnp.maximum(v, 0.0) + jnp.log(1.0 + jnp.exp(-jnp.abs(v)))


def _head_body(ap_ref, an_ref, ddst_ref, b1_ref, wd_ref, out_ref):
    b1 = b1_ref[...]

    def chunk(i):
        rin = lax.rsqrt(
            jnp.maximum(ddst_ref[pl.ds(i * RB, RB), :][:, :1], 1.0))
        pos = ap_ref[pl.ds(i * RB, RB), :] * rin + b1
        neg = an_ref[pl.ds(i * RB, RB), :] * rin + b1
        rowid = lax.broadcasted_iota(jnp.int32, (RB, 1), 0) + i * RB
        m = (rowid < N).astype(jnp.float32)
        return pos, neg, m

    def body1(i, colsum):
        pos, _, m = chunk(i)
        return colsum + jnp.sum(pos * m, axis=0, keepdims=True)

    colsum = lax.fori_loop(0, NBLK, body1, jnp.zeros((1, HID), jnp.float32))
    summary = 1.0 / (1.0 + jnp.exp(-colsum / N))
    wd = wd_ref[...]
    ws = lax.dot_general(summary, wd, (((1,), (1,)), ((), ())),
                         preferred_element_type=jnp.float32)

    def body2(i, carry):
        l1s, l2s = carry
        pos, neg, m = chunk(i)
        psc = lax.dot_general(pos, ws, (((1,), (1,)), ((), ())),
                              preferred_element_type=jnp.float32)
        nsc = lax.dot_general(neg, ws, (((1,), (1,)), ((), ())),
                              preferred_element_type=jnp.float32)
        l1s = l1s + jnp.sum(_softplus(-psc) * m)
        l2s = l2s + jnp.sum(_softplus(nsc) * m)
        return l1s, l2s

    l1s, l2s = lax.fori_loop(
        0, NBLK, body2, (jnp.float32(0.0), jnp.float32(0.0)))
    out_ref[...] = jnp.reshape((l1s + l2s) / jnp.float32(N), (1, 1))


def _head(ap, an, ddst2d, b1, wd):
    return pl.pallas_call(
        _head_body,
        out_shape=jax.ShapeDtypeStruct((1, 1), jnp.float32),
    )(ap, an, ddst2d, b1, wd)


@jax.jit
def kernel(x, edge_index, W0, b0, W1, b1, Wd, Wc, bc):
    del Wc, bc
    e = edge_index.shape[1]
    unit = NS * CH * BATCH
    e_pad = ((e + unit - 1) // unit) * unit

    src = edge_index[0].astype(jnp.int32)
    dst = edge_index[1].astype(jnp.int32)
    pad = jnp.full((e_pad - e,), N, jnp.int32)
    src_p = jnp.concatenate([src, pad])
    dst_p = jnp.concatenate([dst, pad])
    src2d = jnp.reshape(src_p, (e_pad // CH, CH))
    dst2d = jnp.reshape(dst_p, (e_pad // CH, CH))
    perm = jax.random.permutation(jax.random.key(42), N).astype(jnp.int32)
    perm_p = jnp.concatenate([perm, jnp.arange(N, N_T, dtype=jnp.int32)])
    x_p = jnp.pad(x, ((0, N_T - N), (0, 0)))

    dsrc2d, ddst2d = _make_deg(e_pad)(src2d, dst2d)
    h0 = _enc0(x_p, W0)
    (h0p,) = _make_perm()(h0, perm_p)
    tpos, tneg = _scale(h0, h0p, dsrc2d)
    agg = _make_agg(e_pad)
    acc1p, acc1n = agg(tpos, tneg, src_p, src_p, dst_p)
    up, un = _mid(acc1p, acc1n, ddst2d, dsrc2d, jnp.reshape(b0, (1, HID)), W1)
    acc2p, acc2n = agg(up, un, src_p, src_p, dst_p)
    res = _head(acc2p, acc2n, ddst2d, jnp.reshape(b1, (1, HID)), Wd)
    return res[0, 0]


# speedup vs baseline: 1.2977x; 1.2961x over previous
import jax
import jax.numpy as jnp
from jax import lax
from jax.experimental import pallas as pl
from jax.experimental.pallas import tpu as pltpu
from jax.experimental.pallas import tpu_sc as plsc

N = 10000
IN_F = 768
HID = 128

NC = 2
NS = 16
LANES = 16
CH = 128

N_T = 10240
RPT = N_T // NS
RB = 512
NBLK = N_T // RB


def _fill2d(ref, rows, cols, val):
    def body(i, _):
        r = i // (cols // LANES)
        j = i % (cols // LANES)
        ref[r, pl.ds(j * LANES, LANES)] = jnp.full((LANES,), val, jnp.float32)
        return 0
    lax.fori_loop(0, rows * (cols // LANES), body, 0, unroll=False)


def _fill_iota(ref, n, base):
    def body(i, _):
        ref[pl.ds(i * LANES, LANES)] = lax.iota(jnp.int32, LANES) + base + i * LANES
        return 0
    lax.fori_loop(0, n // LANES, body, 0, unroll=False)


def _zero_own_rows(acc, rv, zb_v, s):
    def zb(i, _):
        _fill_iota(rv, CH, s * RPT + i * CH)
        pltpu.sync_copy(zb_v, acc.at[rv])
        return 0
    lax.fori_loop(0, RPT // CH, zb, 0, unroll=False)


def _readout_own_rows(acc, rv, buf_v, o_h, c_sel, s, sem):
    def ob(i, _):
        r = s * RPT + i * CH
        _fill_iota(rv, CH, r)
        pltpu.async_copy(acc.at[rv], buf_v, sem).wait()
        pltpu.sync_copy(buf_v, o_h.at[pl.ds(r, CH)])
        return 0
    lax.fori_loop(0, RPT // CH, ob, 0, unroll=False)


NB = 2
NB_AGG = 1
BATCH = 32


def _make_deg(e_pad):
    chunks = e_pad // (NS * CH)
    mesh = plsc.VectorSubcoreMesh(core_axis_name="c", subcore_axis_name="s",
                                  num_cores=NC, num_subcores=NS)

    def body(src_h, dst_h, dsrc_h, ddst_h, rv, ones_v, zb_v, acc, sem):
        c = lax.axis_index("c")
        s = lax.axis_index("s")
        _fill2d(ones_v, CH, HID, 1.0)
        _fill2d(zb_v, CH, HID, 0.0)
        _zero_own_rows(acc, rv, zb_v, s)
        plsc.subcore_barrier()

        base0 = s * chunks * CH

        def run(i_h, o_h):
            def cb(i, _):
                base = base0 + i * CH
                pltpu.sync_copy(i_h.at[pl.ds(base, CH)], rv)
                pltpu.sync_copy(ones_v, acc.at[rv], add=True)
                return 0
            lax.fori_loop(0, chunks, cb, 0, unroll=False)
            plsc.subcore_barrier()
            _readout_own_rows(acc, rv, zb_v, o_h, c, s, sem)

        @pl.when(c == 0)
        def _():
            run(src_h, dsrc_h)

        @pl.when(c == 1)
        def _():
            run(dst_h, ddst_h)

    return pl.kernel(
        body,
        out_type=(
            jax.ShapeDtypeStruct((N_T, HID), jnp.float32),
            jax.ShapeDtypeStruct((N_T, HID), jnp.float32),
        ),
        mesh=mesh,
        scratch_types=(
            pltpu.VMEM((CH,), jnp.int32),
            pltpu.VMEM((CH, HID), jnp.float32),
            pltpu.VMEM((CH, HID), jnp.float32),
            pltpu.VMEM_SHARED((N_T, HID), jnp.float32),
            pltpu.SemaphoreType.DMA,
        ),
    )


def _make_agg(e_pad):
    chunks = e_pad // (NS * CH)
    mesh = plsc.VectorSubcoreMesh(core_axis_name="c", subcore_axis_name="s",
                                  num_cores=NC, num_subcores=NS)

    def body(t0_h, t1_h, i0_h, i1_h, dst_h, out0_h, out1_h,
             iv, ov, rv, rows_v, zb_v, acc, sem):
        c = lax.axis_index("c")
        s = lax.axis_index("s")
        _fill2d(zb_v, CH, HID, 0.0)
        _zero_own_rows(acc, rv, zb_v, s)
        plsc.subcore_barrier()

        base0 = s * chunks * CH

        def run(t_h, i_h, o_h):
            def cb(i, _):
                base = base0 + i * CH
                pltpu.sync_copy(i_h.at[pl.ds(base, CH)], iv)
                pltpu.sync_copy(dst_h.at[pl.ds(base, CH)], ov)
                pltpu.async_copy(t_h.at[iv], rows_v, sem).wait()
                pltpu.sync_copy(rows_v, acc.at[ov], add=True)
                return 0
            lax.fori_loop(0, chunks, cb, 0, unroll=False)
            plsc.subcore_barrier()
            _readout_own_rows(acc, rv, zb_v, o_h, c, s, sem)

        @pl.when(c == 0)
        def _():
            run(t0_h, i0_h, out0_h)

        @pl.when(c == 1)
        def _():
            run(t1_h, i1_h, out1_h)

    return pl.kernel(
        body,
        out_type=(
            jax.ShapeDtypeStruct((N_T, HID), jnp.float32),
            jax.ShapeDtypeStruct((N_T, HID), jnp.float32),
        ),
        mesh=mesh,
        scratch_types=(
            pltpu.VMEM((CH,), jnp.int32),
            pltpu.VMEM((CH,), jnp.int32),
            pltpu.VMEM((CH,), jnp.int32),
            pltpu.VMEM((CH, HID), jnp.float32),
            pltpu.VMEM((CH, HID), jnp.float32),
            pltpu.VMEM_SHARED((N_T, HID), jnp.float32),
            pltpu.SemaphoreType.DMA,
        ),
    )


def _make_perm():
    nch = N_T // CH
    nw = NC * NS
    mesh = plsc.VectorSubcoreMesh(core_axis_name="c", subcore_axis_name="s",
                                  num_cores=NC, num_subcores=NS)

    def body(t_h, pidx_h, out_h, iv, rows_v, sem):
        c = lax.axis_index("c")
        s = lax.axis_index("s")
        w = c * NS + s

        def cb(k, _):
            ch = w + nw * k

            @pl.when(ch < nch)
            def _():
                base = ch * CH
                pltpu.sync_copy(pidx_h.at[pl.ds(base, CH)], iv)
                pltpu.async_copy(t_h.at[iv], rows_v, sem).wait()
                pltpu.sync_copy(rows_v, out_h.at[pl.ds(base, CH)])
            return 0
        lax.fori_loop(0, (nch + nw - 1) // nw, cb, 0, unroll=False)

    return pl.kernel(
        body,
        out_type=(jax.ShapeDtypeStruct((N_T, HID), jnp.float32),),
        mesh=mesh,
        scratch_types=(
            pltpu.VMEM((CH,), jnp.int32),
            pltpu.VMEM((CH, HID), jnp.float32),
            pltpu.SemaphoreType.DMA,
        ),
    )


def _enc0_body(x_ref, w_ref, h_ref):
    h_ref[...] = jnp.dot(x_ref[...], w_ref[...],
                         preferred_element_type=jnp.float32)


def _enc0(x_p, w0):
    return pl.pallas_call(
        _enc0_body,
        grid=(NBLK,),
        in_specs=[
            pl.BlockSpec((RB, IN_F), lambda i: (i, 0)),
            pl.BlockSpec((IN_F, HID), lambda i: (0, 0)),
        ],
        out_specs=pl.BlockSpec((RB, HID), lambda i: (i, 0)),
        out_shape=jax.ShapeDtypeStruct((N_T, HID), jnp.float32),
    )(x_p, w0)


def _scale_body(h_ref, hp_ref, dsrc_ref, tpos_ref, tneg_ref):
    r_out = lax.rsqrt(jnp.maximum(dsrc_ref[...][:, :1], 1.0))
    tpos_ref[...] = h_ref[...] * r_out
    tneg_ref[...] = hp_ref[...] * r_out


def _scale(h0, h0p, dsrc2d):
    return pl.pallas_call(
        _scale_body,
        grid=(NBLK,),
        in_specs=[
            pl.BlockSpec((RB, HID), lambda i: (i, 0)),
            pl.BlockSpec((RB, HID), lambda i: (i, 0)),
            pl.BlockSpec((RB, HID), lambda i: (i, 0)),
        ],
        out_specs=[
            pl.BlockSpec((RB, HID), lambda i: (i, 0)),
            pl.BlockSpec((RB, HID), lambda i: (i, 0)),
        ],
        out_shape=[
            jax.ShapeDtypeStruct((N_T, HID), jnp.float32),
            jax.ShapeDtypeStruct((N_T, HID), jnp.float32),
        ],
    )(h0, h0p, dsrc2d)


def _mid_body(ap_ref, an_ref, ddst_ref, dsrc_ref, b0_ref, w1_ref,
              up_ref, un_ref):
    rin = lax.rsqrt(jnp.maximum(ddst_ref[...][:, :1], 1.0))
    rout = lax.rsqrt(jnp.maximum(dsrc_ref[...][:, :1], 1.0))
    b0 = b0_ref[...]
    w1 = w1_ref[...]
    zp = jnp.maximum(ap_ref[...] * rin + b0, 0.0)
    zn = jnp.maximum(an_ref[...] * rin + b0, 0.0)
    up_ref[...] = jnp.dot(zp, w1, preferred_element_type=jnp.float32) * rout
    un_ref[...] = jnp.dot(zn, w1, preferred_element_type=jnp.float32) * rout


def _mid(ap, an, ddst2d, dsrc2d, b0, w1):
    return pl.pallas_call(
        _mid_body,
        grid=(NBLK,),
        in_specs=[
            pl.BlockSpec((RB, HID), lambda i: (i, 0)),
            pl.BlockSpec((RB, HID), lambda i: (i, 0)),
            pl.BlockSpec((RB, HID), lambda i: (i, 0)),
            pl.BlockSpec((RB, HID), lambda i: (i, 0)),
            pl.BlockSpec((1, HID), lambda i: (0, 0)),
            pl.BlockSpec((HID, HID), lambda i: (0, 0)),
        ],
        out_specs=[
            pl.BlockSpec((RB, HID), lambda i: (i, 0)),
            pl.BlockSpec((RB, HID), lambda i: (i, 0)),
        ],
        out_shape=[
            jax.ShapeDtypeStruct((N_T, HID), jnp.float32),
            jax.ShapeDtypeStruct((N_T, HID), jnp.float32),
        ],
    )(ap, an, ddst2d, dsrc2d, b0, w1)


def _softplus(v):
    return jnp.maximum(v, 0.0) + jnp.log(1.0 + jnp.exp(-jnp.abs(v)))


def _head_body(ap_ref, an_ref, ddst_ref, b1_ref, wd_ref, out_ref):
    b1 = b1_ref[...]

    def chunk(i):
        rin = lax.rsqrt(
            jnp.maximum(ddst_ref[pl.ds(i * RB, RB), :][:, :1], 1.0))
        pos = ap_ref[pl.ds(i * RB, RB), :] * rin + b1
        neg = an_ref[pl.ds(i * RB, RB), :] * rin + b1
        rowid = lax.broadcasted_iota(jnp.int32, (RB, 1), 0) + i * RB
        m = (rowid < N).astype(jnp.float32)
        return pos, neg, m

    def body1(i, colsum):
        pos, _, m = chunk(i)
        return colsum + jnp.sum(pos * m, axis=0, keepdims=True)

    colsum = lax.fori_loop(0, NBLK, body1, jnp.zeros((1, HID), jnp.float32))
    summary = 1.0 / (1.0 + jnp.exp(-colsum / N))
    wd = wd_ref[...]
    ws = lax.dot_general(summary, wd, (((1,), (1,)), ((), ())),
                         preferred_element_type=jnp.float32)

    def body2(i, carry):
        l1s, l2s = carry
        pos, neg, m = chunk(i)
        psc = lax.dot_general(pos, ws, (((1,), (1,)), ((), ())),
                              preferred_element_type=jnp.float32)
        nsc = lax.dot_general(neg, ws, (((1,), (1,)), ((), ())),
                              preferred_element_type=jnp.float32)
        l1s = l1s + jnp.sum(_softplus(-psc) * m)
        l2s = l2s + jnp.sum(_softplus(nsc) * m)
        return l1s, l2s

    l1s, l2s = lax.fori_loop(
        0, NBLK, body2, (jnp.float32(0.0), jnp.float32(0.0)))
    out_ref[...] = jnp.reshape((l1s + l2s) / jnp.float32(N), (1, 1))


def _head(ap, an, ddst2d, b1, wd):
    return pl.pallas_call(
        _head_body,
        out_shape=jax.ShapeDtypeStruct((1, 1), jnp.float32),
    )(ap, an, ddst2d, b1, wd)


@jax.jit
def kernel(x, edge_index, W0, b0, W1, b1, Wd, Wc, bc):
    del Wc, bc
    e = edge_index.shape[1]
    unit = NS * CH
    e_pad = ((e + unit - 1) // unit) * unit

    src = edge_index[0].astype(jnp.int32)
    dst = edge_index[1].astype(jnp.int32)
    pad = jnp.full((e_pad - e,), N, jnp.int32)
    src_p = jnp.concatenate([src, pad])
    dst_p = jnp.concatenate([dst, pad])
    perm = jax.random.permutation(jax.random.key(42), N).astype(jnp.int32)
    perm_p = jnp.concatenate([perm, jnp.arange(N, N_T, dtype=jnp.int32)])
    x_p = jnp.pad(x, ((0, N_T - N), (0, 0)))

    dsrc2d, ddst2d = _make_deg(e_pad)(src_p, dst_p)
    h0 = _enc0(x_p, W0)
    (h0p,) = _make_perm()(h0, perm_p)
    tpos, tneg = _scale(h0, h0p, dsrc2d)
    agg = _make_agg(e_pad)
    acc1p, acc1n = agg(tpos, tneg, src_p, src_p, dst_p)
    up, un = _mid(acc1p, acc1n, ddst2d, dsrc2d, jnp.reshape(b0, (1, HID)), W1)
    acc2p, acc2n = agg(up, un, src_p, src_p, dst_p)
    res = _head(acc2p, acc2n, ddst2d, jnp.reshape(b1, (1, HID)), Wd)
    return res[0, 0]


# speedup vs baseline: 1.3491x; 1.0396x over previous
import jax
import jax.numpy as jnp
from jax import lax
from jax.experimental import pallas as pl
from jax.experimental.pallas import tpu as pltpu
from jax.experimental.pallas import tpu_sc as plsc

N = 10000
IN_F = 768
HID = 128

NC = 2
NS = 16
LANES = 16
CH = 128

N_T = 10240
RPT = N_T // NS
RB = 512
NBLK = N_T // RB


def _fill2d(ref, rows, cols, val):
    def body(i, _):
        r = i // (cols // LANES)
        j = i % (cols // LANES)
        ref[r, pl.ds(j * LANES, LANES)] = jnp.full((LANES,), val, jnp.float32)
        return 0
    lax.fori_loop(0, rows * (cols // LANES), body, 0, unroll=False)


def _fill_iota(ref, n, base):
    def body(i, _):
        ref[pl.ds(i * LANES, LANES)] = lax.iota(jnp.int32, LANES) + base + i * LANES
        return 0
    lax.fori_loop(0, n // LANES, body, 0, unroll=False)


def _zero_own_rows(acc, rv, zb_v, s):
    def zb(i, _):
        _fill_iota(rv, CH, s * RPT + i * CH)
        pltpu.sync_copy(zb_v, acc.at[rv])
        return 0
    lax.fori_loop(0, RPT // CH, zb, 0, unroll=False)


def _readout_own_rows(acc, rv, buf_v, o_h, c_sel, s, sem):
    def ob(i, _):
        r = s * RPT + i * CH
        _fill_iota(rv, CH, r)
        pltpu.async_copy(acc.at[rv], buf_v, sem).wait()
        pltpu.sync_copy(buf_v, o_h.at[pl.ds(r, CH)])
        return 0
    lax.fori_loop(0, RPT // CH, ob, 0, unroll=False)


NB = 2
NB_AGG = 1
BATCH = 32


def _make_deg(e_pad):
    chunks = e_pad // (NS * CH)
    mesh = plsc.VectorSubcoreMesh(core_axis_name="c", subcore_axis_name="s",
                                  num_cores=NC, num_subcores=NS)

    def body(src_h, dst_h, dsrc_h, ddst_h, rv, ones_v, zb_v, acc, sem):
        c = lax.axis_index("c")
        s = lax.axis_index("s")
        _fill2d(ones_v, CH, HID, 1.0)
        _fill2d(zb_v, CH, HID, 0.0)
        _zero_own_rows(acc, rv, zb_v, s)
        plsc.subcore_barrier()

        base0 = s * chunks * CH

        def run(i_h, o_h):
            def cb(i, _):
                base = base0 + i * CH
                pltpu.sync_copy(i_h.at[pl.ds(base, CH)], rv)
                pltpu.sync_copy(ones_v, acc.at[rv], add=True)
                return 0
            lax.fori_loop(0, chunks, cb, 0, unroll=False)
            plsc.subcore_barrier()
            _readout_own_rows(acc, rv, zb_v, o_h, c, s, sem)

        @pl.when(c == 0)
        def _():
            run(src_h, dsrc_h)

        @pl.when(c == 1)
        def _():
            run(dst_h, ddst_h)

    return pl.kernel(
        body,
        out_type=(
            jax.ShapeDtypeStruct((N_T, HID), jnp.float32),
            jax.ShapeDtypeStruct((N_T, HID), jnp.float32),
        ),
        mesh=mesh,
        scratch_types=(
            pltpu.VMEM((CH,), jnp.int32),
            pltpu.VMEM((CH, HID), jnp.float32),
            pltpu.VMEM((CH, HID), jnp.float32),
            pltpu.VMEM_SHARED((N_T, HID), jnp.float32),
            pltpu.SemaphoreType.DMA,
        ),
    )


def _make_agg(e_pad):
    chunks = e_pad // (NS * CH)
    mesh = plsc.VectorSubcoreMesh(core_axis_name="c", subcore_axis_name="s",
                                  num_cores=NC, num_subcores=NS)

    def body(t0_h, t1_h, i0_h, i1_h, dst_h, out0_h, out1_h,
             iv0, ov0, iv1, ov1, rv, r0, r1, acc, s0, s1):
        c = lax.axis_index("c")
        s = lax.axis_index("s")
        rows = [r0, r1]
        ivs = [iv0, iv1]
        ovs = [ov0, ov1]
        sems = [s0, s1]
        _fill2d(r0, CH, HID, 0.0)
        _zero_own_rows(acc, rv, r0, s)
        plsc.subcore_barrier()

        base0 = s * chunks * CH

        def run(t_h, i_h, o_h):
            def grp(g, _):
                i0 = g * 2
                descs = []
                for b in range(2):
                    base = base0 + (i0 + b) * CH
                    pltpu.sync_copy(i_h.at[pl.ds(base, CH)], ivs[b])
                    pltpu.sync_copy(dst_h.at[pl.ds(base, CH)], ovs[b])
                    descs.append(
                        pltpu.async_copy(t_h.at[ivs[b]], rows[b], sems[b]))
                for b in range(2):
                    descs[b].wait()
                    pltpu.sync_copy(rows[b], acc.at[ovs[b]], add=True)
                return 0
            lax.fori_loop(0, chunks // 2, grp, 0, unroll=False)
            plsc.subcore_barrier()
            _readout_own_rows(acc, rv, r0, o_h, c, s, sems[0])

        @pl.when(c == 0)
        def _():
            run(t0_h, i0_h, out0_h)

        @pl.when(c == 1)
        def _():
            run(t1_h, i1_h, out1_h)

    return pl.kernel(
        body,
        out_type=(
            jax.ShapeDtypeStruct((N_T, HID), jnp.float32),
            jax.ShapeDtypeStruct((N_T, HID), jnp.float32),
        ),
        mesh=mesh,
        scratch_types=(
            pltpu.VMEM((CH,), jnp.int32),
            pltpu.VMEM((CH,), jnp.int32),
            pltpu.VMEM((CH,), jnp.int32),
            pltpu.VMEM((CH,), jnp.int32),
            pltpu.VMEM((CH,), jnp.int32),
            pltpu.VMEM((CH, HID), jnp.float32),
            pltpu.VMEM((CH, HID), jnp.float32),
            pltpu.VMEM_SHARED((N_T, HID), jnp.float32),
            pltpu.SemaphoreType.DMA,
            pltpu.SemaphoreType.DMA,
        ),
    )


def _make_perm():
    nch = N_T // CH
    nw = NC * NS
    mesh = plsc.VectorSubcoreMesh(core_axis_name="c", subcore_axis_name="s",
                                  num_cores=NC, num_subcores=NS)

    def body(t_h, pidx_h, out_h, iv, rows_v, sem):
        c = lax.axis_index("c")
        s = lax.axis_index("s")
        w = c * NS + s

        def cb(k, _):
            ch = w + nw * k

            @pl.when(ch < nch)
            def _():
                base = ch * CH
                pltpu.sync_copy(pidx_h.at[pl.ds(base, CH)], iv)
                pltpu.async_copy(t_h.at[iv], rows_v, sem).wait()
                pltpu.sync_copy(rows_v, out_h.at[pl.ds(base, CH)])
            return 0
        lax.fori_loop(0, (nch + nw - 1) // nw, cb, 0, unroll=False)

    return pl.kernel(
        body,
        out_type=(jax.ShapeDtypeStruct((N_T, HID), jnp.float32),),
        mesh=mesh,
        scratch_types=(
            pltpu.VMEM((CH,), jnp.int32),
            pltpu.VMEM((CH, HID), jnp.float32),
            pltpu.SemaphoreType.DMA,
        ),
    )


def _enc0_body(x_ref, w_ref, h_ref):
    h_ref[...] = jnp.dot(x_ref[...], w_ref[...],
                         preferred_element_type=jnp.float32)


def _enc0(x_p, w0):
    return pl.pallas_call(
        _enc0_body,
        grid=(NBLK,),
        in_specs=[
            pl.BlockSpec((RB, IN_F), lambda i: (i, 0)),
            pl.BlockSpec((IN_F, HID), lambda i: (0, 0)),
        ],
        out_specs=pl.BlockSpec((RB, HID), lambda i: (i, 0)),
        out_shape=jax.ShapeDtypeStruct((N_T, HID), jnp.float32),
    )(x_p, w0)


def _scale_body(h_ref, hp_ref, dsrc_ref, tpos_ref, tneg_ref):
    r_out = lax.rsqrt(jnp.maximum(dsrc_ref[...][:, :1], 1.0))
    tpos_ref[...] = h_ref[...] * r_out
    tneg_ref[...] = hp_ref[...] * r_out


def _scale(h0, h0p, dsrc2d):
    return pl.pallas_call(
        _scale_body,
        grid=(NBLK,),
        in_specs=[
            pl.BlockSpec((RB, HID), lambda i: (i, 0)),
            pl.BlockSpec((RB, HID), lambda i: (i, 0)),
            pl.BlockSpec((RB, HID), lambda i: (i, 0)),
        ],
        out_specs=[
            pl.BlockSpec((RB, HID), lambda i: (i, 0)),
            pl.BlockSpec((RB, HID), lambda i: (i, 0)),
        ],
        out_shape=[
            jax.ShapeDtypeStruct((N_T, HID), jnp.float32),
            jax.ShapeDtypeStruct((N_T, HID), jnp.float32),
        ],
    )(h0, h0p, dsrc2d)


def _mid_body(ap_ref, an_ref, ddst_ref, dsrc_ref, b0_ref, w1_ref,
              up_ref, un_ref):
    rin = lax.rsqrt(jnp.maximum(ddst_ref[...][:, :1], 1.0))
    rout = lax.rsqrt(jnp.maximum(dsrc_ref[...][:, :1], 1.0))
    b0 = b0_ref[...]
    w1 = w1_ref[...]
    zp = jnp.maximum(ap_ref[...] * rin + b0, 0.0)
    zn = jnp.maximum(an_ref[...] * rin + b0, 0.0)
    up_ref[...] = jnp.dot(zp, w1, preferred_element_type=jnp.float32) * rout
    un_ref[...] = jnp.dot(zn, w1, preferred_element_type=jnp.float32) * rout


def _mid(ap, an, ddst2d, dsrc2d, b0, w1):
    return pl.pallas_call(
        _mid_body,
        grid=(NBLK,),
        in_specs=[
            pl.BlockSpec((RB, HID), lambda i: (i, 0)),
            pl.BlockSpec((RB, HID), lambda i: (i, 0)),
            pl.BlockSpec((RB, HID), lambda i: (i, 0)),
            pl.BlockSpec((RB, HID), lambda i: (i, 0)),
            pl.BlockSpec((1, HID), lambda i: (0, 0)),
            pl.BlockSpec((HID, HID), lambda i: (0, 0)),
        ],
        out_specs=[
            pl.BlockSpec((RB, HID), lambda i: (i, 0)),
            pl.BlockSpec((RB, HID), lambda i: (i, 0)),
        ],
        out_shape=[
            jax.ShapeDtypeStruct((N_T, HID), jnp.float32),
            jax.ShapeDtypeStruct((N_T, HID), jnp.float32),
        ],
    )(ap, an, ddst2d, dsrc2d, b0, w1)


def _softplus(v):
    return jnp.maximum(v, 0.0) + jnp.log(1.0 + jnp.exp(-jnp.abs(v)))


def _head_body(ap_ref, an_ref, ddst_ref, b1_ref, wd_ref, out_ref):
    b1 = b1_ref[...]

    def chunk(i):
        rin = lax.rsqrt(
            jnp.maximum(ddst_ref[pl.ds(i * RB, RB), :][:, :1], 1.0))
        pos = ap_ref[pl.ds(i * RB, RB), :] * rin + b1
        neg = an_ref[pl.ds(i * RB, RB), :] * rin + b1
        rowid = lax.broadcasted_iota(jnp.int32, (RB, 1), 0) + i * RB
        m = (rowid < N).astype(jnp.float32)
        return pos, neg, m

    def body1(i, colsum):
        pos, _, m = chunk(i)
        return colsum + jnp.sum(pos * m, axis=0, keepdims=True)

    colsum = lax.fori_loop(0, NBLK, body1, jnp.zeros((1, HID), jnp.float32))
    summary = 1.0 / (1.0 + jnp.exp(-colsum / N))
    wd = wd_ref[...]
    ws = lax.dot_general(summary, wd, (((1,), (1,)), ((), ())),
                         preferred_element_type=jnp.float32)

    def body2(i, carry):
        l1s, l2s = carry
        pos, neg, m = chunk(i)
        psc = lax.dot_general(pos, ws, (((1,), (1,)), ((), ())),
                              preferred_element_type=jnp.float32)
        nsc = lax.dot_general(neg, ws, (((1,), (1,)), ((), ())),
                              preferred_element_type=jnp.float32)
        l1s = l1s + jnp.sum(_softplus(-psc) * m)
        l2s = l2s + jnp.sum(_softplus(nsc) * m)
        return l1s, l2s

    l1s, l2s = lax.fori_loop(
        0, NBLK, body2, (jnp.float32(0.0), jnp.float32(0.0)))
    out_ref[...] = jnp.reshape((l1s + l2s) / jnp.float32(N), (1, 1))


def _head(ap, an, ddst2d, b1, wd):
    return pl.pallas_call(
        _head_body,
        out_shape=jax.ShapeDtypeStruct((1, 1), jnp.float32),
    )(ap, an, ddst2d, b1, wd)


@jax.jit
def kernel(x, edge_index, W0, b0, W1, b1, Wd, Wc, bc):
    del Wc, bc
    e = edge_index.shape[1]
    unit = NS * CH * 2
    e_pad = ((e + unit - 1) // unit) * unit

    src = edge_index[0].astype(jnp.int32)
    dst = edge_index[1].astype(jnp.int32)
    pad = jnp.full((e_pad - e,), N, jnp.int32)
    src_p = jnp.concatenate([src, pad])
    dst_p = jnp.concatenate([dst, pad])
    perm = jax.random.permutation(jax.random.key(42), N).astype(jnp.int32)
    perm_p = jnp.concatenate([perm, jnp.arange(N, N_T, dtype=jnp.int32)])
    x_p = jnp.pad(x, ((0, N_T - N), (0, 0)))

    dsrc2d, ddst2d = _make_deg(e_pad)(src_p, dst_p)
    h0 = _enc0(x_p, W0)
    (h0p,) = _make_perm()(h0, perm_p)
    tpos, tneg = _scale(h0, h0p, dsrc2d)
    agg = _make_agg(e_pad)
    acc1p, acc1n = agg(tpos, tneg, src_p, src_p, dst_p)
    up, un = _mid(acc1p, acc1n, ddst2d, dsrc2d, jnp.reshape(b0, (1, HID)), W1)
    acc2p, acc2n = agg(up, un, src_p, src_p, dst_p)
    res = _head(acc2p, acc2n, ddst2d, jnp.reshape(b1, (1, HID)), Wd)
    return res[0, 0]


# speedup vs baseline: 1.4615x; 1.0833x over previous
import jax
import jax.numpy as jnp
import numpy as np
from jax import lax
from jax.experimental import pallas as pl
from jax.experimental.pallas import tpu as pltpu
from jax.experimental.pallas import tpu_sc as plsc

N = 10000
IN_F = 768
HID = 128

NC = 2
NS = 16
LANES = 16
CH = 128

N_T = 10240
RPT = N_T // NS
RB = 512
NBLK = N_T // RB
CHE = 80

_PERM_P = np.concatenate([
    np.asarray(jax.random.permutation(jax.random.key(42), N)),
    np.arange(N, N_T),
]).astype(np.int32)


def _fill2d(ref, rows, cols, val):
    def body(i, _):
        r = i // (cols // LANES)
        j = i % (cols // LANES)
        ref[r, pl.ds(j * LANES, LANES)] = jnp.full((LANES,), val, jnp.float32)
        return 0
    lax.fori_loop(0, rows * (cols // LANES), body, 0, unroll=False)


def _fill_iota(ref, n, base):
    def body(i, _):
        ref[pl.ds(i * LANES, LANES)] = lax.iota(jnp.int32, LANES) + base + i * LANES
        return 0
    lax.fori_loop(0, n // LANES, body, 0, unroll=False)


def _zero_own_rows(acc, rv, zb_v, s, w=CH):
    def zb(i, _):
        _fill_iota(rv, w, s * RPT + i * w)
        pltpu.sync_copy(zb_v, acc.at[rv])
        return 0
    lax.fori_loop(0, RPT // w, zb, 0, unroll=False)


def _readout_own_rows(acc, rv, buf_v, o_h, c_sel, s, sem, w=CH):
    def ob(i, _):
        r = s * RPT + i * w
        _fill_iota(rv, w, r)
        pltpu.async_copy(acc.at[rv], buf_v, sem).wait()
        pltpu.sync_copy(buf_v, o_h.at[pl.ds(r, w)])
        return 0
    lax.fori_loop(0, RPT // w, ob, 0, unroll=False)


NB = 2
NB_AGG = 1
BATCH = 32


def _make_deg(e_pad):
    chunks = e_pad // (NS * CH)
    mesh = plsc.VectorSubcoreMesh(core_axis_name="c", subcore_axis_name="s",
                                  num_cores=NC, num_subcores=NS)

    def body(src_h, dst_h, dsrc_h, ddst_h, rv, ones_v, zb_v, acc, sem):
        c = lax.axis_index("c")
        s = lax.axis_index("s")
        _fill2d(ones_v, CH, HID, 1.0)
        _fill2d(zb_v, CH, HID, 0.0)
        _zero_own_rows(acc, rv, zb_v, s)
        plsc.subcore_barrier()

        base0 = s * chunks * CH

        def run(i_h, o_h):
            def cb(i, _):
                base = base0 + i * CH
                pltpu.sync_copy(i_h.at[pl.ds(base, CH)], rv)
                pltpu.sync_copy(ones_v, acc.at[rv], add=True)
                return 0
            lax.fori_loop(0, chunks, cb, 0, unroll=False)
            plsc.subcore_barrier()
            _readout_own_rows(acc, rv, zb_v, o_h, c, s, sem)

        @pl.when(c == 0)
        def _():
            run(src_h, dsrc_h)

        @pl.when(c == 1)
        def _():
            run(dst_h, ddst_h)

    return pl.kernel(
        body,
        out_type=(
            jax.ShapeDtypeStruct((N_T, HID), jnp.float32),
            jax.ShapeDtypeStruct((N_T, HID), jnp.float32),
        ),
        mesh=mesh,
        scratch_types=(
            pltpu.VMEM((CH,), jnp.int32),
            pltpu.VMEM((CH, HID), jnp.float32),
            pltpu.VMEM((CH, HID), jnp.float32),
            pltpu.VMEM_SHARED((N_T, HID), jnp.float32),
            pltpu.SemaphoreType.DMA,
        ),
    )


def _make_agg(e_pad):
    chunks = e_pad // (NS * CHE)
    mesh = plsc.VectorSubcoreMesh(core_axis_name="c", subcore_axis_name="s",
                                  num_cores=NC, num_subcores=NS)

    def body(t0_h, t1_h, i0_h, i1_h, dst_h, out0_h, out1_h,
             iv0, ov0, iv1, ov1, iv2, ov2, rv, r0, r1, r2, acc,
             s0, s1, s2):
        c = lax.axis_index("c")
        s = lax.axis_index("s")
        rows = [r0, r1, r2]
        ivs = [iv0, iv1, iv2]
        ovs = [ov0, ov1, ov2]
        sems = [s0, s1, s2]
        _fill2d(r0, CHE, HID, 0.0)
        _zero_own_rows(acc, rv, r0, s, w=CHE)
        plsc.subcore_barrier()

        base0 = s * chunks * CHE

        def run(t_h, i_h, o_h):
            def grp(g, _):
                i0 = g * 3
                descs = []
                for b in range(3):
                    base = base0 + (i0 + b) * CHE
                    pltpu.sync_copy(i_h.at[pl.ds(base, CHE)], ivs[b])
                    pltpu.sync_copy(dst_h.at[pl.ds(base, CHE)], ovs[b])
                    descs.append(
                        pltpu.async_copy(t_h.at[ivs[b]], rows[b], sems[b]))
                for b in range(3):
                    descs[b].wait()
                    pltpu.sync_copy(rows[b], acc.at[ovs[b]], add=True)
                return 0
            lax.fori_loop(0, chunks // 3, grp, 0, unroll=False)
            plsc.subcore_barrier()
            _readout_own_rows(acc, rv, r0, o_h, c, s, sems[0], w=CHE)

        @pl.when(c == 0)
        def _():
            run(t0_h, i0_h, out0_h)

        @pl.when(c == 1)
        def _():
            run(t1_h, i1_h, out1_h)

    return pl.kernel(
        body,
        out_type=(
            jax.ShapeDtypeStruct((N_T, HID), jnp.float32),
            jax.ShapeDtypeStruct((N_T, HID), jnp.float32),
        ),
        mesh=mesh,
        scratch_types=(
            pltpu.VMEM((CHE,), jnp.int32),
            pltpu.VMEM((CHE,), jnp.int32),
            pltpu.VMEM((CHE,), jnp.int32),
            pltpu.VMEM((CHE,), jnp.int32),
            pltpu.VMEM((CHE,), jnp.int32),
            pltpu.VMEM((CHE,), jnp.int32),
            pltpu.VMEM((CHE,), jnp.int32),
            pltpu.VMEM((CHE, HID), jnp.float32),
            pltpu.VMEM((CHE, HID), jnp.float32),
            pltpu.VMEM((CHE, HID), jnp.float32),
            pltpu.VMEM_SHARED((N_T, HID), jnp.float32),
            pltpu.SemaphoreType.DMA,
            pltpu.SemaphoreType.DMA,
            pltpu.SemaphoreType.DMA,
        ),
    )


def _make_perm():
    nch = N_T // CH
    nw = NC * NS
    mesh = plsc.VectorSubcoreMesh(core_axis_name="c", subcore_axis_name="s",
                                  num_cores=NC, num_subcores=NS)

    def body(t_h, pidx_h, out_h, iv, rows_v, sem):
        c = lax.axis_index("c")
        s = lax.axis_index("s")
        w = c * NS + s

        def cb(k, _):
            ch = w + nw * k

            @pl.when(ch < nch)
            def _():
                base = ch * CH
                pltpu.sync_copy(pidx_h.at[pl.ds(base, CH)], iv)
                pltpu.async_copy(t_h.at[iv], rows_v, sem).wait()
                pltpu.sync_copy(rows_v, out_h.at[pl.ds(base, CH)])
            return 0
        lax.fori_loop(0, (nch + nw - 1) // nw, cb, 0, unroll=False)

    return pl.kernel(
        body,
        out_type=(jax.ShapeDtypeStruct((N_T, HID), jnp.float32),),
        mesh=mesh,
        scratch_types=(
            pltpu.VMEM((CH,), jnp.int32),
            pltpu.VMEM((CH, HID), jnp.float32),
            pltpu.SemaphoreType.DMA,
        ),
    )


def _enc0_body(x_ref, w_ref, h_ref):
    h_ref[...] = jnp.dot(x_ref[...], w_ref[...],
                         preferred_element_type=jnp.float32)


def _enc0(x_p, w0):
    return pl.pallas_call(
        _enc0_body,
        grid=(NBLK,),
        in_specs=[
            pl.BlockSpec((RB, IN_F), lambda i: (i, 0)),
            pl.BlockSpec((IN_F, HID), lambda i: (0, 0)),
        ],
        out_specs=pl.BlockSpec((RB, HID), lambda i: (i, 0)),
        out_shape=jax.ShapeDtypeStruct((N_T, HID), jnp.float32),
    )(x_p, w0)


def _scale_body(h_ref, hp_ref, dsrc_ref, tpos_ref, tneg_ref):
    r_out = lax.rsqrt(jnp.maximum(dsrc_ref[...][:, :1], 1.0))
    tpos_ref[...] = h_ref[...] * r_out
    tneg_ref[...] = hp_ref[...] * r_out


def _scale(h0, h0p, dsrc2d):
    return pl.pallas_call(
        _scale_body,
        grid=(NBLK,),
        in_specs=[
            pl.BlockSpec((RB, HID), lambda i: (i, 0)),
            pl.BlockSpec((RB, HID), lambda i: (i, 0)),
            pl.BlockSpec((RB, HID), lambda i: (i, 0)),
        ],
        out_specs=[
            pl.BlockSpec((RB, HID), lambda i: (i, 0)),
            pl.BlockSpec((RB, HID), lambda i: (i, 0)),
        ],
        out_shape=[
            jax.ShapeDtypeStruct((N_T, HID), jnp.float32),
            jax.ShapeDtypeStruct((N_T, HID), jnp.float32),
        ],
    )(h0, h0p, dsrc2d)


def _mid_body(ap_ref, an_ref, ddst_ref, dsrc_ref, b0_ref, w1_ref,
              up_ref, un_ref):
    rin = lax.rsqrt(jnp.maximum(ddst_ref[...][:, :1], 1.0))
    rout = lax.rsqrt(jnp.maximum(dsrc_ref[...][:, :1], 1.0))
    b0 = b0_ref[...]
    w1 = w1_ref[...]
    zp = jnp.maximum(ap_ref[...] * rin + b0, 0.0)
    zn = jnp.maximum(an_ref[...] * rin + b0, 0.0)
    up_ref[...] = jnp.dot(zp, w1, preferred_element_type=jnp.float32) * rout
    un_ref[...] = jnp.dot(zn, w1, preferred_element_type=jnp.float32) * rout


def _mid(ap, an, ddst2d, dsrc2d, b0, w1):
    return pl.pallas_call(
        _mid_body,
        grid=(NBLK,),
        in_specs=[
            pl.BlockSpec((RB, HID), lambda i: (i, 0)),
            pl.BlockSpec((RB, HID), lambda i: (i, 0)),
            pl.BlockSpec((RB, HID), lambda i: (i, 0)),
            pl.BlockSpec((RB, HID), lambda i: (i, 0)),
            pl.BlockSpec((1, HID), lambda i: (0, 0)),
            pl.BlockSpec((HID, HID), lambda i: (0, 0)),
        ],
        out_specs=[
            pl.BlockSpec((RB, HID), lambda i: (i, 0)),
            pl.BlockSpec((RB, HID), lambda i: (i, 0)),
        ],
        out_shape=[
            jax.ShapeDtypeStruct((N_T, HID), jnp.float32),
            jax.ShapeDtypeStruct((N_T, HID), jnp.float32),
        ],
    )(ap, an, ddst2d, dsrc2d, b0, w1)


def _softplus(v):
    return jnp.maximum(v, 0.0) + jnp.log(1.0 + jnp.exp(-jnp.abs(v)))


def _head_body(ap_ref, an_ref, ddst_ref, b1_ref, wd_ref, out_ref):
    b1 = b1_ref[...]

    def chunk(i):
        rin = lax.rsqrt(
            jnp.maximum(ddst_ref[pl.ds(i * RB, RB), :][:, :1], 1.0))
        pos = ap_ref[pl.ds(i * RB, RB), :] * rin + b1
        neg = an_ref[pl.ds(i * RB, RB), :] * rin + b1
        rowid = lax.broadcasted_iota(jnp.int32, (RB, 1), 0) + i * RB
        m = (rowid < N).astype(jnp.float32)
        return pos, neg, m

    def body1(i, colsum):
        pos, _, m = chunk(i)
        return colsum + jnp.sum(pos * m, axis=0, keepdims=True)

    colsum = lax.fori_loop(0, NBLK, body1, jnp.zeros((1, HID), jnp.float32))
    summary = 1.0 / (1.0 + jnp.exp(-colsum / N))
    wd = wd_ref[...]
    ws = lax.dot_general(summary, wd, (((1,), (1,)), ((), ())),
                         preferred_element_type=jnp.float32)

    def body2(i, carry):
        l1s, l2s = carry
        pos, neg, m = chunk(i)
        psc = lax.dot_general(pos, ws, (((1,), (1,)), ((), ())),
                              preferred_element_type=jnp.float32)
        nsc = lax.dot_general(neg, ws, (((1,), (1,)), ((), ())),
                              preferred_element_type=jnp.float32)
        l1s = l1s + jnp.sum(_softplus(-psc) * m)
        l2s = l2s + jnp.sum(_softplus(nsc) * m)
        return l1s, l2s

    l1s, l2s = lax.fori_loop(
        0, NBLK, body2, (jnp.float32(0.0), jnp.float32(0.0)))
    out_ref[...] = jnp.reshape((l1s + l2s) / jnp.float32(N), (1, 1))


def _head(ap, an, ddst2d, b1, wd):
    return pl.pallas_call(
        _head_body,
        out_shape=jax.ShapeDtypeStruct((1, 1), jnp.float32),
    )(ap, an, ddst2d, b1, wd)


@jax.jit
def kernel(x, edge_index, W0, b0, W1, b1, Wd, Wc, bc):
    del Wc, bc
    e = edge_index.shape[1]
    unit = NS * CHE * 3
    e_pad = ((e + unit - 1) // unit) * unit

    src = edge_index[0].astype(jnp.int32)
    dst = edge_index[1].astype(jnp.int32)
    pad = jnp.full((e_pad - e,), N, jnp.int32)
    src_p = jnp.concatenate([src, pad])
    dst_p = jnp.concatenate([dst, pad])
    perm_p = jnp.asarray(_PERM_P)
    x_p = jnp.pad(x, ((0, N_T - N), (0, 0)))

    dsrc2d, ddst2d = _make_deg(e_pad)(src_p, dst_p)
    h0 = _enc0(x_p, W0)
    (h0p,) = _make_perm()(h0, perm_p)
    tpos, tneg = _scale(h0, h0p, dsrc2d)
    agg = _make_agg(e_pad)
    acc1p, acc1n = agg(tpos, tneg, src_p, src_p, dst_p)
    up, un = _mid(acc1p, acc1n, ddst2d, dsrc2d, jnp.reshape(b0, (1, HID)), W1)
    acc2p, acc2n = agg(up, un, src_p, src_p, dst_p)
    res = _head(acc2p, acc2n, ddst2d, jnp.reshape(b1, (1, HID)), Wd)
    return res[0, 0]
